# baseline (speedup 1.0000x reference)
import functools

import jax
import jax.numpy as jnp
from jax import lax
from jax.experimental import pallas as pl
from jax.experimental.pallas import tpu as pltpu
from jax.experimental.pallas import tpu_sc as plsc

B = 64
H = 768
L = 24576
K = 64

ENC_BL = 2048
DEC_BL = 2048

NV = L // 16
GRP = 4
NG = NV // GRP
SPAN = 96
NSP = NV // SPAN
CAP = L + 16

_u32 = jnp.uint32
_i32 = jnp.int32


def _sortable(v):
    ub = lax.bitcast_convert_type(v, _u32)
    return jnp.where((ub >> 31) == 1, ~ub, ub | _u32(0x80000000))


def _encode_body(x_ref, pb_ref, w_ref, lb_ref, out_ref):
    xm = x_ref[...] - pb_ref[...]
    acc = lax.dot_general(
        xm, w_ref[...], (((1,), (1,)), ((), ())),
        preferred_element_type=jnp.float32)
    out_ref[...] = acc + lb_ref[...]


def _ssum(v_i32):
    return lax.reduce_sum(v_i32, axes=(0,))


def _sc_topk_fn():
    mesh = plsc.VectorSubcoreMesh(core_axis_name="c", subcore_axis_name="s")

    @functools.partial(
        pl.kernel, mesh=mesh,
        compiler_params=pltpu.CompilerParams(needs_layout_passes=False),
        out_type=(jax.ShapeDtypeStruct((B, 16), _i32),
                  jax.ShapeDtypeStruct((B, 16), _i32)),
        scratch_types=[
            pltpu.VMEM((L,), jnp.float32),
            pltpu.VMEM((L,), jnp.float32),
            pltpu.VMEM((NG * 16,), _u32),
            pltpu.VMEM((NSP * 16,), _u32),
            pltpu.VMEM((CAP,), _u32),
            pltpu.VMEM((CAP,), _i32),
            pltpu.VMEM((16,), _i32),
            pltpu.VMEM((16,), _i32),
            pltpu.SemaphoreType.DMA,
            pltpu.SemaphoreType.DMA,
        ],
    )
    def sc_topk(pre_hbm, t_hbm, c_hbm, rowa_v, rowb_v, gmax_v, m2_v,
                ck_v, ci_v, to_v, co_v, sema, semb):
        c = lax.axis_index("c")
        s = lax.axis_index("s")
        wid = s * 2 + c
        r0 = wid * 2

        cpa = pltpu.async_copy(pre_hbm.at[r0], rowa_v, sema)
        cpb = pltpu.async_copy(pre_hbm.at[r0 + 1], rowb_v, semb)

        lanes = lax.iota(_i32, 16)

        def popcnt(mask):
            return plsc.all_reduce_population_count(mask)[0]

        def do_row(row_v, rr, cp):
            cp.wait()

            def p1_span(sp, _):
                def p1_grp(gg, m2):
                    g = sp * (SPAN // GRP) + gg
                    m = jnp.zeros((16,), _u32)
                    for t in range(GRP):
                        kk = _sortable(row_v[pl.ds((g * GRP + t) * 16, 16)])
                        m = jnp.maximum(m, kk)
                    gmax_v[pl.ds(g * 16, 16)] = m
                    return jnp.maximum(m2, m)
                m2 = lax.fori_loop(0, SPAN // GRP, p1_grp,
                                   jnp.zeros((16,), _u32), unroll=2)
                m2_v[pl.ds(sp * 16, 16)] = m2
                return 0

            lax.fori_loop(0, NSP, p1_span, 0)

            def t0_bit(i, T):
                cand = T | (_u32(1) << (_u32(31) - i.astype(_u32)))

                def acc(j, cv):
                    return cv + (m2_v[pl.ds(j * 16, 16)] >= cand).astype(_i32)

                cnt = _ssum(lax.fori_loop(0, NSP, acc,
                                          jnp.zeros((16,), _i32), unroll=4))
                return jnp.where(cnt >= K, cand, T)

            t0 = lax.fori_loop(0, 32, t0_bit, _u32(0))

            def p2(g, off):
                gm = gmax_v[pl.ds(g * 16, 16)]
                hit = popcnt(gm >= t0) > 0

                def collect(off2):
                    for t in range(GRP):
                        base = (g * GRP + t) * 16
                        kk = _sortable(row_v[pl.ds(base, 16)])
                        m = kk >= t0
                        pc = popcnt(m)
                        plsc.store_compressed(ck_v.at[pl.ds(off2, 16)], kk, mask=m)
                        plsc.store_compressed(ci_v.at[pl.ds(off2, 16)],
                                              lanes + base, mask=m)
                        off2 = off2 + pc
                    return off2

                return lax.cond(hit, collect, lambda o: o, off)

            ncand = lax.fori_loop(0, NG, p2, _i32(0))

            ck_v[pl.ds(ncand, 16)] = jnp.zeros((16,), _u32)
            ci_v[pl.ds(ncand, 16)] = jnp.full((16,), L, _i32)
            nv = (ncand + 15) // 16

            def p3_bit(i, T):
                cand = T | (_u32(1) << (_u32(31) - i.astype(_u32)))

                def acc(j, cv):
                    return cv + (ck_v[pl.ds(j * 16, 16)] >= cand).astype(_i32)

                cnt = _ssum(lax.fori_loop(0, nv, acc,
                                          jnp.zeros((16,), _i32)))
                return jnp.where(cnt >= K, cand, T)

            tkey = lax.fori_loop(0, 32, p3_bit, _u32(0))

            def acc_gt(j, cv):
                return cv + (ck_v[pl.ds(j * 16, 16)] > tkey).astype(_i32)

            cnt_gt = _ssum(lax.fori_loop(0, nv, acc_gt,
                                         jnp.zeros((16,), _i32)))
            need = K - cnt_gt

            def p4(j, st):
                acc2, cidx = st
                tie = (ck_v[pl.ds(j * 16, 16)] == tkey)
                ti = tie.astype(_i32)
                cnt = _ssum(ti)
                cs = plsc.cumsum(ti)
                want = need - acc2
                m = tie & (cs == want)
                lane = lax.reduce_min(jnp.where(m, lanes, 16), axes=(0,))
                hit = (acc2 < need) & (lane < 16)
                idxv = _ssum(jnp.where(lanes == lane, ci_v[pl.ds(j * 16, 16)],
                                       0))
                cidx = jnp.where(hit, idxv + 1, cidx)
                return (acc2 + cnt, cidx)

            _, cfin = lax.fori_loop(0, nv, p4, (_i32(0), _i32(0)))

            to_v[...] = jnp.full((16,), lax.bitcast_convert_type(tkey, _i32),
                                 _i32)
            co_v[...] = jnp.full((16,), cfin, _i32)
            pltpu.sync_copy(to_v, t_hbm.at[rr])
            pltpu.sync_copy(co_v, c_hbm.at[rr])

        do_row(rowa_v, r0, cpa)
        do_row(rowb_v, r0 + 1, cpb)

    return sc_topk


def _decode_body(pre_ref, w_ref, t_ref, c_ref, pb_ref, lat_ref, xhat_ref):
    j = pl.program_id(0)
    pre = pre_ref[...]
    key = _sortable(pre)
    T = lax.bitcast_convert_type(t_ref[:, :1], _u32)
    C = c_ref[:, :1]
    idx = lax.broadcasted_iota(_i32, (B, DEC_BL), 1) + j * DEC_BL
    keep = (key > T) | ((key == T) & (idx < C))
    lat = jnp.where(keep, pre, 0.0)
    lat_ref[:, 0, :] = lat
    part = lax.dot_general(
        lat, w_ref[...], (((1,), (1,)), ((), ())),
        preferred_element_type=jnp.float32)

    @pl.when(j == 0)
    def _():
        xhat_ref[:, 0, :] = jnp.broadcast_to(pb_ref[...], (B, H))

    xhat_ref[:, 0, :] += part


@jax.jit
def kernel(x, W_enc, W_dec, pre_bias, latent_bias):
    x2d = x.reshape(B, H)
    pb = pre_bias.reshape(1, H)
    lb = latent_bias.reshape(1, L)

    pre = pl.pallas_call(
        _encode_body,
        grid=(L // ENC_BL,),
        in_specs=[
            pl.BlockSpec((B, H), lambda j: (0, 0)),
            pl.BlockSpec((1, H), lambda j: (0, 0)),
            pl.BlockSpec((ENC_BL, H), lambda j: (j, 0)),
            pl.BlockSpec((1, ENC_BL), lambda j: (0, j)),
        ],
        out_specs=pl.BlockSpec((B, ENC_BL), lambda j: (0, j)),
        out_shape=jax.ShapeDtypeStruct((B, L), jnp.float32),
    )(x2d, pb, W_enc, lb)

    T, C = _sc_topk_fn()(pre)

    latents, x_hat = pl.pallas_call(
        _decode_body,
        grid=(L // DEC_BL,),
        in_specs=[
            pl.BlockSpec((B, DEC_BL), lambda j: (0, j)),
            pl.BlockSpec((H, DEC_BL), lambda j: (0, j)),
            pl.BlockSpec((B, 16), lambda j: (0, 0)),
            pl.BlockSpec((B, 16), lambda j: (0, 0)),
            pl.BlockSpec((1, H), lambda j: (0, 0)),
        ],
        out_specs=(pl.BlockSpec((B, 1, DEC_BL), lambda j: (0, 0, j)),
                   pl.BlockSpec((B, 1, H), lambda j: (0, 0, 0))),
        out_shape=(jax.ShapeDtypeStruct((B, 1, L), jnp.float32),
                   jax.ShapeDtypeStruct((B, 1, H), jnp.float32)),
    )(pre, W_dec, T, C, pb)

    return latents, x_hat


# speedup vs baseline: 1.0148x; 1.0148x over previous
import functools

import jax
import jax.numpy as jnp
from jax import lax
from jax.experimental import pallas as pl
from jax.experimental.pallas import tpu as pltpu
from jax.experimental.pallas import tpu_sc as plsc

B = 64
H = 768
L = 24576
K = 64

ENC_BL = 2048
DEC_BL = 2048

NV = L // 16
SPAN = 96
NSP = NV // SPAN
CAP = L + 16

_u32 = jnp.uint32
_i32 = jnp.int32


def _sortable(v):
    ub = lax.bitcast_convert_type(v, _u32)
    return jnp.where((ub >> 31) == 1, ~ub, ub | _u32(0x80000000))


def _encode_body(x_ref, pb_ref, w_ref, lb_ref, out_ref):
    xm = x_ref[...] - pb_ref[...]
    acc = lax.dot_general(
        xm, w_ref[...], (((1,), (1,)), ((), ())),
        preferred_element_type=jnp.float32)
    out_ref[...] = acc + lb_ref[...]


def _ssum(v_i32):
    return lax.reduce_sum(v_i32, axes=(0,))


def _sc_topk_fn():
    mesh = plsc.VectorSubcoreMesh(core_axis_name="c", subcore_axis_name="s")

    @functools.partial(
        pl.kernel, mesh=mesh,
        compiler_params=pltpu.CompilerParams(needs_layout_passes=False),
        out_type=(jax.ShapeDtypeStruct((B, 16), _i32),
                  jax.ShapeDtypeStruct((B, 16), _i32)),
        scratch_types=[
            pltpu.VMEM((L,), jnp.float32),
            pltpu.VMEM((L,), _u32),
            pltpu.VMEM((NSP * 16,), _u32),
            pltpu.VMEM((CAP,), _u32),
            pltpu.VMEM((CAP,), _i32),
            pltpu.VMEM((16,), _i32),
            pltpu.VMEM((16,), _i32),
            pltpu.SemaphoreType.DMA,
        ],
    )
    def sc_topk(pre_hbm, t_hbm, c_hbm, row_v, keys_v, m2_v,
                ck_v, ci_v, to_v, co_v, sem):
        c = lax.axis_index("c")
        s = lax.axis_index("s")
        wid = s * 2 + c
        r0 = wid * 2

        lanes = lax.iota(_i32, 16)

        def popcnt(mask):
            return plsc.all_reduce_population_count(mask)[0]

        pltpu.async_copy(pre_hbm.at[r0], row_v, sem).wait()

        def do_row(rr, next_r, prefetch):
            def p1_span(sp, _):
                def p1_v(i, m2):
                    j = sp * SPAN + i
                    kk = _sortable(row_v[pl.ds(j * 16, 16)])
                    keys_v[pl.ds(j * 16, 16)] = kk
                    return jnp.maximum(m2, kk)
                m2 = lax.fori_loop(0, SPAN, p1_v,
                                   jnp.zeros((16,), _u32), unroll=4)
                m2_v[pl.ds(sp * 16, 16)] = m2
                return 0

            lax.fori_loop(0, NSP, p1_span, 0)

            nxt = (pltpu.async_copy(pre_hbm.at[next_r], row_v, sem)
                   if prefetch else None)

            def t0_bit(i, T):
                cand = T | (_u32(1) << (_u32(31) - i.astype(_u32)))

                def acc(j, cv):
                    return cv + (m2_v[pl.ds(j * 16, 16)] >= cand).astype(_i32)

                cnt = _ssum(lax.fori_loop(0, NSP, acc,
                                          jnp.zeros((16,), _i32), unroll=4))
                return jnp.where(cnt >= K, cand, T)

            t0 = lax.fori_loop(0, 32, t0_bit, _u32(0))

            def p2(j, off):
                kk = keys_v[pl.ds(j * 16, 16)]
                m = kk >= t0
                pc = popcnt(m)
                plsc.store_compressed(ck_v.at[pl.ds(off, 16)], kk, mask=m)
                plsc.store_compressed(ci_v.at[pl.ds(off, 16)],
                                      lanes + j * 16, mask=m)
                return off + pc

            ncand = lax.fori_loop(0, NV, p2, _i32(0), unroll=4)

            ck_v[pl.ds(ncand, 16)] = jnp.zeros((16,), _u32)
            ci_v[pl.ds(ncand, 16)] = jnp.full((16,), L, _i32)
            nv = (ncand + 15) // 16

            def p3_bit(i, T):
                cand = T | (_u32(1) << (_u32(31) - i.astype(_u32)))

                def acc(j, cv):
                    return cv + (ck_v[pl.ds(j * 16, 16)] >= cand).astype(_i32)

                cnt = _ssum(lax.fori_loop(0, nv, acc,
                                          jnp.zeros((16,), _i32)))
                return jnp.where(cnt >= K, cand, T)

            tkey = lax.fori_loop(0, 32, p3_bit, _u32(0))

            def acc_gt(j, cv):
                return cv + (ck_v[pl.ds(j * 16, 16)] > tkey).astype(_i32)

            cnt_gt = _ssum(lax.fori_loop(0, nv, acc_gt,
                                         jnp.zeros((16,), _i32)))
            need = K - cnt_gt

            def p4(j, st):
                acc2, cidx = st
                tie = (ck_v[pl.ds(j * 16, 16)] == tkey)
                ti = tie.astype(_i32)
                cnt = _ssum(ti)
                cs = plsc.cumsum(ti)
                want = need - acc2
                m = tie & (cs == want)
                lane = lax.reduce_min(jnp.where(m, lanes, 16), axes=(0,))
                hit = (acc2 < need) & (lane < 16)
                idxv = _ssum(jnp.where(lanes == lane,
                                       ci_v[pl.ds(j * 16, 16)], 0))
                cidx = jnp.where(hit, idxv + 1, cidx)
                return (acc2 + cnt, cidx)

            _, cfin = lax.fori_loop(0, nv, p4, (_i32(0), _i32(0)))

            to_v[...] = jnp.full((16,), lax.bitcast_convert_type(tkey, _i32),
                                 _i32)
            co_v[...] = jnp.full((16,), cfin, _i32)
            pltpu.sync_copy(to_v, t_hbm.at[rr])
            pltpu.sync_copy(co_v, c_hbm.at[rr])
            return nxt

        nxt = do_row(r0, r0 + 1, True)
        nxt.wait()
        do_row(r0 + 1, r0 + 1, False)

    return sc_topk


def _decode_body(pre_ref, w_ref, t_ref, c_ref, pb_ref, lat_ref, xhat_ref):
    j = pl.program_id(0)
    pre = pre_ref[...]
    key = _sortable(pre)
    T = lax.bitcast_convert_type(t_ref[:, :1], _u32)
    C = c_ref[:, :1]
    idx = lax.broadcasted_iota(_i32, (B, DEC_BL), 1) + j * DEC_BL
    keep = (key > T) | ((key == T) & (idx < C))
    lat = jnp.where(keep, pre, 0.0)
    lat_ref[:, 0, :] = lat
    part = lax.dot_general(
        lat, w_ref[...], (((1,), (1,)), ((), ())),
        preferred_element_type=jnp.float32)

    @pl.when(j == 0)
    def _():
        xhat_ref[:, 0, :] = jnp.broadcast_to(pb_ref[...], (B, H))

    xhat_ref[:, 0, :] += part


@jax.jit
def kernel(x, W_enc, W_dec, pre_bias, latent_bias):
    x2d = x.reshape(B, H)
    pb = pre_bias.reshape(1, H)
    lb = latent_bias.reshape(1, L)

    pre = pl.pallas_call(
        _encode_body,
        grid=(L // ENC_BL,),
        in_specs=[
            pl.BlockSpec((B, H), lambda j: (0, 0)),
            pl.BlockSpec((1, H), lambda j: (0, 0)),
            pl.BlockSpec((ENC_BL, H), lambda j: (j, 0)),
            pl.BlockSpec((1, ENC_BL), lambda j: (0, j)),
        ],
        out_specs=pl.BlockSpec((B, ENC_BL), lambda j: (0, j)),
        out_shape=jax.ShapeDtypeStruct((B, L), jnp.float32),
    )(x2d, pb, W_enc, lb)

    T, C = _sc_topk_fn()(pre)

    latents, x_hat = pl.pallas_call(
        _decode_body,
        grid=(L // DEC_BL,),
        in_specs=[
            pl.BlockSpec((B, DEC_BL), lambda j: (0, j)),
            pl.BlockSpec((H, DEC_BL), lambda j: (0, j)),
            pl.BlockSpec((B, 16), lambda j: (0, 0)),
            pl.BlockSpec((B, 16), lambda j: (0, 0)),
            pl.BlockSpec((1, H), lambda j: (0, 0)),
        ],
        out_specs=(pl.BlockSpec((B, 1, DEC_BL), lambda j: (0, 0, j)),
                   pl.BlockSpec((B, 1, H), lambda j: (0, 0, 0))),
        out_shape=(jax.ShapeDtypeStruct((B, 1, L), jnp.float32),
                   jax.ShapeDtypeStruct((B, 1, H), jnp.float32)),
    )(pre, W_dec, T, C, pb)

    return latents, x_hat


# speedup vs baseline: 1.1972x; 1.1797x over previous
import functools

import jax
import jax.numpy as jnp
from jax import lax
from jax.experimental import pallas as pl
from jax.experimental.pallas import tpu as pltpu
from jax.experimental.pallas import tpu_sc as plsc

B = 64
H = 768
L = 24576
K = 64

ENC_BL = 2048
DEC_BL = 2048

NV = L // 16
SPAN = 96
NSP = NV // SPAN
CAP = L + 16

_u32 = jnp.uint32
_i32 = jnp.int32


def _sortable(v):
    ub = lax.bitcast_convert_type(v, _u32)
    return jnp.where((ub >> 31) == 1, ~ub, ub | _u32(0x80000000))


def _encode_body(x_ref, pb_ref, w_ref, lb_ref, out_ref):
    xm = x_ref[...] - pb_ref[...]
    acc = lax.dot_general(
        xm, w_ref[...], (((1,), (1,)), ((), ())),
        preferred_element_type=jnp.float32)
    out_ref[...] = acc + lb_ref[...]


def _ssum(v_i32):
    return lax.reduce_sum(v_i32, axes=(0,))


def _sc_topk_fn():
    mesh = plsc.VectorSubcoreMesh(core_axis_name="c", subcore_axis_name="s")

    @functools.partial(
        pl.kernel, mesh=mesh,
        compiler_params=pltpu.CompilerParams(needs_layout_passes=False),
        out_type=(jax.ShapeDtypeStruct((B, 16), _i32),
                  jax.ShapeDtypeStruct((B, 16), _i32)),
        scratch_types=[
            pltpu.VMEM((L,), jnp.float32),
            pltpu.VMEM((L,), _u32),
            pltpu.VMEM((NSP * 16,), _u32),
            pltpu.VMEM((CAP,), _u32),
            pltpu.VMEM((CAP,), _i32),
            pltpu.VMEM((16,), _i32),
            pltpu.VMEM((16,), _i32),
            pltpu.SemaphoreType.DMA,
        ],
    )
    def sc_topk(pre_hbm, t_hbm, c_hbm, row_v, keys_v, m2_v,
                ck_v, ci_v, to_v, co_v, sem):
        c = lax.axis_index("c")
        s = lax.axis_index("s")
        wid = s * 2 + c
        r0 = wid * 2

        lanes = lax.iota(_i32, 16)

        def popcnt(mask):
            return plsc.all_reduce_population_count(mask)[0]

        pltpu.async_copy(pre_hbm.at[r0], row_v, sem).wait()

        def do_row(rr, next_r, prefetch):
            def p1_span(sp, _):
                def p1_v(i, m2):
                    j = sp * SPAN + i
                    kk = _sortable(row_v[pl.ds(j * 16, 16)])
                    keys_v[pl.ds(j * 16, 16)] = kk
                    return jnp.maximum(m2, kk)
                m2 = lax.fori_loop(0, SPAN, p1_v,
                                   jnp.zeros((16,), _u32), unroll=4)
                m2_v[pl.ds(sp * 16, 16)] = m2
                return 0

            lax.fori_loop(0, NSP, p1_span, 0)

            nxt = (pltpu.async_copy(pre_hbm.at[next_r], row_v, sem)
                   if prefetch else None)

            def t0_bit(i, T):
                cand = T | (_u32(1) << (_u32(31) - i.astype(_u32)))

                def acc(j, cv):
                    return cv + (m2_v[pl.ds(j * 16, 16)] >= cand).astype(_i32)

                cnt = _ssum(lax.fori_loop(0, NSP, acc,
                                          jnp.zeros((16,), _i32), unroll=4))
                return jnp.where(cnt >= K, cand, T)

            t0 = lax.fori_loop(0, 32, t0_bit, _u32(0))

            PB = 8

            def p2(jb, off):
                kks = []
                ms = []
                pcs = []
                for t in range(PB):
                    kk = keys_v[pl.ds((jb * PB + t) * 16, 16)]
                    m = kk >= t0
                    kks.append(kk)
                    ms.append(m)
                    pcs.append(popcnt(m))
                for t in range(PB):
                    plsc.store_compressed(ck_v.at[pl.ds(off, 16)], kks[t],
                                          mask=ms[t])
                    plsc.store_compressed(ci_v.at[pl.ds(off, 16)],
                                          lanes + (jb * PB + t) * 16,
                                          mask=ms[t])
                    off = off + pcs[t]
                return off

            ncand = lax.fori_loop(0, NV // PB, p2, _i32(0))

            ck_v[pl.ds(ncand, 16)] = jnp.zeros((16,), _u32)
            ci_v[pl.ds(ncand, 16)] = jnp.full((16,), L, _i32)
            nv = (ncand + 15) // 16

            def p3_bit(i, T):
                cand = T | (_u32(1) << (_u32(31) - i.astype(_u32)))

                def acc(j, cv):
                    return cv + (ck_v[pl.ds(j * 16, 16)] >= cand).astype(_i32)

                cnt = _ssum(lax.fori_loop(0, nv, acc,
                                          jnp.zeros((16,), _i32)))
                return jnp.where(cnt >= K, cand, T)

            tkey = lax.fori_loop(0, 32, p3_bit, _u32(0))

            def acc_gt(j, cv):
                return cv + (ck_v[pl.ds(j * 16, 16)] > tkey).astype(_i32)

            cnt_gt = _ssum(lax.fori_loop(0, nv, acc_gt,
                                         jnp.zeros((16,), _i32)))
            need = K - cnt_gt

            def p4(j, st):
                acc2, cidx = st
                tie = (ck_v[pl.ds(j * 16, 16)] == tkey)
                ti = tie.astype(_i32)
                cnt = _ssum(ti)
                cs = plsc.cumsum(ti)
                want = need - acc2
                m = tie & (cs == want)
                lane = lax.reduce_min(jnp.where(m, lanes, 16), axes=(0,))
                hit = (acc2 < need) & (lane < 16)
                idxv = _ssum(jnp.where(lanes == lane,
                                       ci_v[pl.ds(j * 16, 16)], 0))
                cidx = jnp.where(hit, idxv + 1, cidx)
                return (acc2 + cnt, cidx)

            _, cfin = lax.fori_loop(0, nv, p4, (_i32(0), _i32(0)))

            to_v[...] = jnp.full((16,), lax.bitcast_convert_type(tkey, _i32),
                                 _i32)
            co_v[...] = jnp.full((16,), cfin, _i32)
            pltpu.sync_copy(to_v, t_hbm.at[rr])
            pltpu.sync_copy(co_v, c_hbm.at[rr])
            return nxt

        nxt = do_row(r0, r0 + 1, True)
        nxt.wait()
        do_row(r0 + 1, r0 + 1, False)

    return sc_topk


def _decode_body(pre_ref, w_ref, t_ref, c_ref, pb_ref, lat_ref, xhat_ref):
    j = pl.program_id(0)
    pre = pre_ref[...]
    key = _sortable(pre)
    T = lax.bitcast_convert_type(t_ref[:, :1], _u32)
    C = c_ref[:, :1]
    idx = lax.broadcasted_iota(_i32, (B, DEC_BL), 1) + j * DEC_BL
    keep = (key > T) | ((key == T) & (idx < C))
    lat = jnp.where(keep, pre, 0.0)
    lat_ref[:, 0, :] = lat
    part = lax.dot_general(
        lat, w_ref[...], (((1,), (1,)), ((), ())),
        preferred_element_type=jnp.float32)

    @pl.when(j == 0)
    def _():
        xhat_ref[:, 0, :] = jnp.broadcast_to(pb_ref[...], (B, H))

    xhat_ref[:, 0, :] += part


@jax.jit
def kernel(x, W_enc, W_dec, pre_bias, latent_bias):
    x2d = x.reshape(B, H)
    pb = pre_bias.reshape(1, H)
    lb = latent_bias.reshape(1, L)

    pre = pl.pallas_call(
        _encode_body,
        grid=(L // ENC_BL,),
        in_specs=[
            pl.BlockSpec((B, H), lambda j: (0, 0)),
            pl.BlockSpec((1, H), lambda j: (0, 0)),
            pl.BlockSpec((ENC_BL, H), lambda j: (j, 0)),
            pl.BlockSpec((1, ENC_BL), lambda j: (0, j)),
        ],
        out_specs=pl.BlockSpec((B, ENC_BL), lambda j: (0, j)),
        out_shape=jax.ShapeDtypeStruct((B, L), jnp.float32),
    )(x2d, pb, W_enc, lb)

    T, C = _sc_topk_fn()(pre)

    latents, x_hat = pl.pallas_call(
        _decode_body,
        grid=(L // DEC_BL,),
        in_specs=[
            pl.BlockSpec((B, DEC_BL), lambda j: (0, j)),
            pl.BlockSpec((H, DEC_BL), lambda j: (0, j)),
            pl.BlockSpec((B, 16), lambda j: (0, 0)),
            pl.BlockSpec((B, 16), lambda j: (0, 0)),
            pl.BlockSpec((1, H), lambda j: (0, 0)),
        ],
        out_specs=(pl.BlockSpec((B, 1, DEC_BL), lambda j: (0, 0, j)),
                   pl.BlockSpec((B, 1, H), lambda j: (0, 0, 0))),
        out_shape=(jax.ShapeDtypeStruct((B, 1, L), jnp.float32),
                   jax.ShapeDtypeStruct((B, 1, H), jnp.float32)),
    )(pre, W_dec, T, C, pb)

    return latents, x_hat


# speedup vs baseline: 1.2864x; 1.0745x over previous
import functools

import jax
import jax.numpy as jnp
from jax import lax
from jax.experimental import pallas as pl
from jax.experimental.pallas import tpu as pltpu
from jax.experimental.pallas import tpu_sc as plsc

B = 64
H = 768
L = 24576
K = 64

ENC_BL = 2048
DEC_BL = 2048

NV = L // 16
SPAN = 96
NSP = NV // SPAN
CAP = L + 16

_u32 = jnp.uint32
_i32 = jnp.int32


def _sortable(v):
    ub = lax.bitcast_convert_type(v, _u32)
    return jnp.where((ub >> 31) == 1, ~ub, ub | _u32(0x80000000))


def _encode_body(x_ref, pb_ref, w_ref, lb_ref, out_ref):
    xm = x_ref[...] - pb_ref[...]
    acc = lax.dot_general(
        xm, w_ref[...], (((1,), (1,)), ((), ())),
        preferred_element_type=jnp.float32)
    out_ref[...] = acc + lb_ref[...]


def _ssum(v_i32):
    return lax.reduce_sum(v_i32, axes=(0,))


def _sc_topk_fn():
    mesh = plsc.VectorSubcoreMesh(core_axis_name="c", subcore_axis_name="s")

    @functools.partial(
        pl.kernel, mesh=mesh,
        compiler_params=pltpu.CompilerParams(needs_layout_passes=False),
        out_type=(jax.ShapeDtypeStruct((B, 16), _i32),
                  jax.ShapeDtypeStruct((B, 16), _i32)),
        scratch_types=[
            pltpu.VMEM((L,), jnp.float32),
            pltpu.VMEM((L,), _u32),
            pltpu.VMEM((NSP * 16,), _u32),
            pltpu.VMEM((CAP,), _u32),
            pltpu.VMEM((CAP,), _i32),
            pltpu.VMEM((16,), _i32),
            pltpu.VMEM((16,), _i32),
            pltpu.SemaphoreType.DMA,
        ],
    )
    def sc_topk(pre_hbm, t_hbm, c_hbm, row_v, keys_v, m2_v,
                ck_v, ci_v, to_v, co_v, sem):
        c = lax.axis_index("c")
        s = lax.axis_index("s")
        wid = s * 2 + c
        r0 = wid * 2

        lanes = lax.iota(_i32, 16)

        def popcnt(mask):
            return plsc.all_reduce_population_count(mask)[0]

        pltpu.async_copy(pre_hbm.at[r0], row_v, sem).wait()

        def do_row(rr, next_r, prefetch):
            def p1_span(sp, _):
                def p1_v(i, ms):
                    m2a, m2b = ms
                    j = sp * SPAN + i * 2
                    ka = _sortable(row_v[pl.ds(j * 16, 16)])
                    kb = _sortable(row_v[pl.ds((j + 1) * 16, 16)])
                    keys_v[pl.ds(j * 16, 16)] = ka
                    keys_v[pl.ds((j + 1) * 16, 16)] = kb
                    return (jnp.maximum(m2a, ka), jnp.maximum(m2b, kb))
                z = jnp.zeros((16,), _u32)
                m2a, m2b = lax.fori_loop(0, SPAN // 2, p1_v, (z, z),
                                         unroll=4)
                m2_v[pl.ds(sp * 16, 16)] = jnp.maximum(m2a, m2b)
                return 0

            lax.fori_loop(0, NSP, p1_span, 0)

            nxt = (pltpu.async_copy(pre_hbm.at[next_r], row_v, sem)
                   if prefetch else None)

            def t0_bit(i, T):
                cand = T | (_u32(1) << (_u32(31) - i.astype(_u32)))

                def acc(j, cv):
                    return cv + (m2_v[pl.ds(j * 16, 16)] >= cand).astype(_i32)

                cnt = _ssum(lax.fori_loop(0, NSP, acc,
                                          jnp.zeros((16,), _i32), unroll=4))
                return jnp.where(cnt >= K, cand, T)

            t0 = lax.fori_loop(0, 32, t0_bit, _u32(0))

            PB = 16

            def p2(jb, off):
                kks = []
                ms = []
                pcs = []
                for t in range(PB):
                    kk = keys_v[pl.ds((jb * PB + t) * 16, 16)]
                    m = kk >= t0
                    kks.append(kk)
                    ms.append(m)
                    pcs.append(popcnt(m))
                for t in range(PB):
                    plsc.store_compressed(ck_v.at[pl.ds(off, 16)], kks[t],
                                          mask=ms[t])
                    plsc.store_compressed(ci_v.at[pl.ds(off, 16)],
                                          lanes + (jb * PB + t) * 16,
                                          mask=ms[t])
                    off = off + pcs[t]
                return off

            ncand = lax.fori_loop(0, NV // PB, p2, _i32(0))

            ck_v[pl.ds(ncand, 16)] = jnp.zeros((16,), _u32)
            ci_v[pl.ds(ncand, 16)] = jnp.full((16,), L, _i32)
            nv = (ncand + 15) // 16

            def p3_bit(i, T):
                cand = T | (_u32(1) << (_u32(31) - i.astype(_u32)))

                def acc(j, cv):
                    return cv + (ck_v[pl.ds(j * 16, 16)] >= cand).astype(_i32)

                cnt = _ssum(lax.fori_loop(0, nv, acc,
                                          jnp.zeros((16,), _i32)))
                return jnp.where(cnt >= K, cand, T)

            tkey = lax.fori_loop(0, 32, p3_bit, _u32(0))

            def acc_gt(j, cv):
                return cv + (ck_v[pl.ds(j * 16, 16)] > tkey).astype(_i32)

            cnt_gt = _ssum(lax.fori_loop(0, nv, acc_gt,
                                         jnp.zeros((16,), _i32)))
            need = K - cnt_gt

            def p4(j, st):
                acc2, cidx = st
                tie = (ck_v[pl.ds(j * 16, 16)] == tkey)
                ti = tie.astype(_i32)
                cnt = _ssum(ti)
                cs = plsc.cumsum(ti)
                want = need - acc2
                m = tie & (cs == want)
                lane = lax.reduce_min(jnp.where(m, lanes, 16), axes=(0,))
                hit = (acc2 < need) & (lane < 16)
                idxv = _ssum(jnp.where(lanes == lane,
                                       ci_v[pl.ds(j * 16, 16)], 0))
                cidx = jnp.where(hit, idxv + 1, cidx)
                return (acc2 + cnt, cidx)

            _, cfin = lax.fori_loop(0, nv, p4, (_i32(0), _i32(0)))

            to_v[...] = jnp.full((16,), lax.bitcast_convert_type(tkey, _i32),
                                 _i32)
            co_v[...] = jnp.full((16,), cfin, _i32)
            pltpu.sync_copy(to_v, t_hbm.at[rr])
            pltpu.sync_copy(co_v, c_hbm.at[rr])
            return nxt

        nxt = do_row(r0, r0 + 1, True)
        nxt.wait()
        do_row(r0 + 1, r0 + 1, False)

    return sc_topk


def _decode_body(pre_ref, w_ref, t_ref, c_ref, pb_ref, lat_ref, xhat_ref):
    j = pl.program_id(0)
    pre = pre_ref[...]
    key = _sortable(pre)
    T = lax.bitcast_convert_type(t_ref[:, :1], _u32)
    C = c_ref[:, :1]
    idx = lax.broadcasted_iota(_i32, (B, DEC_BL), 1) + j * DEC_BL
    keep = (key > T) | ((key == T) & (idx < C))
    lat = jnp.where(keep, pre, 0.0)
    lat_ref[:, 0, :] = lat
    part = lax.dot_general(
        lat, w_ref[...], (((1,), (1,)), ((), ())),
        preferred_element_type=jnp.float32)

    @pl.when(j == 0)
    def _():
        xhat_ref[:, 0, :] = jnp.broadcast_to(pb_ref[...], (B, H))

    xhat_ref[:, 0, :] += part


@jax.jit
def kernel(x, W_enc, W_dec, pre_bias, latent_bias):
    x2d = x.reshape(B, H)
    pb = pre_bias.reshape(1, H)
    lb = latent_bias.reshape(1, L)

    pre = pl.pallas_call(
        _encode_body,
        grid=(L // ENC_BL,),
        in_specs=[
            pl.BlockSpec((B, H), lambda j: (0, 0)),
            pl.BlockSpec((1, H), lambda j: (0, 0)),
            pl.BlockSpec((ENC_BL, H), lambda j: (j, 0)),
            pl.BlockSpec((1, ENC_BL), lambda j: (0, j)),
        ],
        out_specs=pl.BlockSpec((B, ENC_BL), lambda j: (0, j)),
        out_shape=jax.ShapeDtypeStruct((B, L), jnp.float32),
    )(x2d, pb, W_enc, lb)

    T, C = _sc_topk_fn()(pre)

    latents, x_hat = pl.pallas_call(
        _decode_body,
        grid=(L // DEC_BL,),
        in_specs=[
            pl.BlockSpec((B, DEC_BL), lambda j: (0, j)),
            pl.BlockSpec((H, DEC_BL), lambda j: (0, j)),
            pl.BlockSpec((B, 16), lambda j: (0, 0)),
            pl.BlockSpec((B, 16), lambda j: (0, 0)),
            pl.BlockSpec((1, H), lambda j: (0, 0)),
        ],
        out_specs=(pl.BlockSpec((B, 1, DEC_BL), lambda j: (0, 0, j)),
                   pl.BlockSpec((B, 1, H), lambda j: (0, 0, 0))),
        out_shape=(jax.ShapeDtypeStruct((B, 1, L), jnp.float32),
                   jax.ShapeDtypeStruct((B, 1, H), jnp.float32)),
    )(pre, W_dec, T, C, pb)

    return latents, x_hat


# speedup vs baseline: 1.2905x; 1.0031x over previous
import functools

import jax
import jax.numpy as jnp
from jax import lax
from jax.experimental import pallas as pl
from jax.experimental.pallas import tpu as pltpu
from jax.experimental.pallas import tpu_sc as plsc

B = 64
H = 768
L = 24576
K = 64

ENC_BL = 4096
DEC_BL = 4096

NV = L // 16
SPAN = 96
NSP = NV // SPAN
CAP = L + 16

_u32 = jnp.uint32
_i32 = jnp.int32


def _sortable(v):
    ub = lax.bitcast_convert_type(v, _u32)
    return jnp.where((ub >> 31) == 1, ~ub, ub | _u32(0x80000000))


def _encode_body(x_ref, pb_ref, w_ref, lb_ref, out_ref):
    xm = x_ref[...] - pb_ref[...]
    acc = lax.dot_general(
        xm, w_ref[...], (((1,), (1,)), ((), ())),
        preferred_element_type=jnp.float32)
    out_ref[...] = acc + lb_ref[...]


def _ssum(v_i32):
    return lax.reduce_sum(v_i32, axes=(0,))


def _sc_topk_fn():
    mesh = plsc.VectorSubcoreMesh(core_axis_name="c", subcore_axis_name="s")

    @functools.partial(
        pl.kernel, mesh=mesh,
        compiler_params=pltpu.CompilerParams(needs_layout_passes=False),
        out_type=(jax.ShapeDtypeStruct((B, 16), _i32),
                  jax.ShapeDtypeStruct((B, 16), _i32)),
        scratch_types=[
            pltpu.VMEM((L,), jnp.float32),
            pltpu.VMEM((L,), _u32),
            pltpu.VMEM((NSP * 16,), _u32),
            pltpu.VMEM((CAP,), _u32),
            pltpu.VMEM((CAP,), _i32),
            pltpu.VMEM((16,), _i32),
            pltpu.VMEM((16,), _i32),
            pltpu.SemaphoreType.DMA,
        ],
    )
    def sc_topk(pre_hbm, t_hbm, c_hbm, row_v, keys_v, m2_v,
                ck_v, ci_v, to_v, co_v, sem):
        c = lax.axis_index("c")
        s = lax.axis_index("s")
        wid = s * 2 + c
        r0 = wid * 2

        lanes = lax.iota(_i32, 16)

        def popcnt(mask):
            return plsc.all_reduce_population_count(mask)[0]

        pltpu.async_copy(pre_hbm.at[r0], row_v, sem).wait()

        def do_row(rr, next_r, prefetch):
            def p1_span(sp, _):
                def p1_v(i, ms):
                    m2a, m2b = ms
                    j = sp * SPAN + i * 2
                    ka = _sortable(row_v[pl.ds(j * 16, 16)])
                    kb = _sortable(row_v[pl.ds((j + 1) * 16, 16)])
                    keys_v[pl.ds(j * 16, 16)] = ka
                    keys_v[pl.ds((j + 1) * 16, 16)] = kb
                    return (jnp.maximum(m2a, ka), jnp.maximum(m2b, kb))
                z = jnp.zeros((16,), _u32)
                m2a, m2b = lax.fori_loop(0, SPAN // 2, p1_v, (z, z),
                                         unroll=4)
                m2_v[pl.ds(sp * 16, 16)] = jnp.maximum(m2a, m2b)
                return 0

            lax.fori_loop(0, NSP, p1_span, 0)

            nxt = (pltpu.async_copy(pre_hbm.at[next_r], row_v, sem)
                   if prefetch else None)

            def t0_bit(i, T):
                cand = T | (_u32(1) << (_u32(31) - i.astype(_u32)))

                def acc(j, cv):
                    return cv + (m2_v[pl.ds(j * 16, 16)] >= cand).astype(_i32)

                cnt = _ssum(lax.fori_loop(0, NSP, acc,
                                          jnp.zeros((16,), _i32), unroll=4))
                return jnp.where(cnt >= K, cand, T)

            t0 = lax.fori_loop(0, 32, t0_bit, _u32(0))

            PB = 16

            def p2(jb, off):
                kks = []
                ms = []
                pcs = []
                for t in range(PB):
                    kk = keys_v[pl.ds((jb * PB + t) * 16, 16)]
                    m = kk >= t0
                    kks.append(kk)
                    ms.append(m)
                    pcs.append(popcnt(m))
                for t in range(PB):
                    plsc.store_compressed(ck_v.at[pl.ds(off, 16)], kks[t],
                                          mask=ms[t])
                    plsc.store_compressed(ci_v.at[pl.ds(off, 16)],
                                          lanes + (jb * PB + t) * 16,
                                          mask=ms[t])
                    off = off + pcs[t]
                return off

            ncand = lax.fori_loop(0, NV // PB, p2, _i32(0))

            ck_v[pl.ds(ncand, 16)] = jnp.zeros((16,), _u32)
            ci_v[pl.ds(ncand, 16)] = jnp.full((16,), L, _i32)
            nv = (ncand + 15) // 16

            def p3_bit(i, T):
                cand = T | (_u32(1) << (_u32(31) - i.astype(_u32)))

                def acc(j, cv):
                    return cv + (ck_v[pl.ds(j * 16, 16)] >= cand).astype(_i32)

                cnt = _ssum(lax.fori_loop(0, nv, acc,
                                          jnp.zeros((16,), _i32)))
                return jnp.where(cnt >= K, cand, T)

            tkey = lax.fori_loop(0, 32, p3_bit, _u32(0))

            def acc_gt(j, cv):
                return cv + (ck_v[pl.ds(j * 16, 16)] > tkey).astype(_i32)

            cnt_gt = _ssum(lax.fori_loop(0, nv, acc_gt,
                                         jnp.zeros((16,), _i32)))
            need = K - cnt_gt

            def p4(j, st):
                acc2, cidx = st
                tie = (ck_v[pl.ds(j * 16, 16)] == tkey)
                ti = tie.astype(_i32)
                cnt = _ssum(ti)
                cs = plsc.cumsum(ti)
                want = need - acc2
                m = tie & (cs == want)
                lane = lax.reduce_min(jnp.where(m, lanes, 16), axes=(0,))
                hit = (acc2 < need) & (lane < 16)
                idxv = _ssum(jnp.where(lanes == lane,
                                       ci_v[pl.ds(j * 16, 16)], 0))
                cidx = jnp.where(hit, idxv + 1, cidx)
                return (acc2 + cnt, cidx)

            _, cfin = lax.fori_loop(0, nv, p4, (_i32(0), _i32(0)))

            to_v[...] = jnp.full((16,), lax.bitcast_convert_type(tkey, _i32),
                                 _i32)
            co_v[...] = jnp.full((16,), cfin, _i32)
            pltpu.sync_copy(to_v, t_hbm.at[rr])
            pltpu.sync_copy(co_v, c_hbm.at[rr])
            return nxt

        nxt = do_row(r0, r0 + 1, True)
        nxt.wait()
        do_row(r0 + 1, r0 + 1, False)

    return sc_topk


def _decode_body(pre_ref, w_ref, t_ref, c_ref, pb_ref, lat_ref, xhat_ref):
    j = pl.program_id(0)
    pre = pre_ref[...]
    key = _sortable(pre)
    T = lax.bitcast_convert_type(t_ref[:, :1], _u32)
    C = c_ref[:, :1]
    idx = lax.broadcasted_iota(_i32, (B, DEC_BL), 1) + j * DEC_BL
    keep = (key > T) | ((key == T) & (idx < C))
    lat = jnp.where(keep, pre, 0.0)
    lat_ref[:, 0, :] = lat
    part = lax.dot_general(
        lat, w_ref[...], (((1,), (1,)), ((), ())),
        preferred_element_type=jnp.float32)

    @pl.when(j == 0)
    def _():
        xhat_ref[:, 0, :] = jnp.broadcast_to(pb_ref[...], (B, H))

    xhat_ref[:, 0, :] += part


@jax.jit
def kernel(x, W_enc, W_dec, pre_bias, latent_bias):
    x2d = x.reshape(B, H)
    pb = pre_bias.reshape(1, H)
    lb = latent_bias.reshape(1, L)

    pre = pl.pallas_call(
        _encode_body,
        grid=(L // ENC_BL,),
        in_specs=[
            pl.BlockSpec((B, H), lambda j: (0, 0)),
            pl.BlockSpec((1, H), lambda j: (0, 0)),
            pl.BlockSpec((ENC_BL, H), lambda j: (j, 0)),
            pl.BlockSpec((1, ENC_BL), lambda j: (0, j)),
        ],
        out_specs=pl.BlockSpec((B, ENC_BL), lambda j: (0, j)),
        out_shape=jax.ShapeDtypeStruct((B, L), jnp.float32),
    )(x2d, pb, W_enc, lb)

    T, C = _sc_topk_fn()(pre)

    latents, x_hat = pl.pallas_call(
        _decode_body,
        grid=(L // DEC_BL,),
        in_specs=[
            pl.BlockSpec((B, DEC_BL), lambda j: (0, j)),
            pl.BlockSpec((H, DEC_BL), lambda j: (0, j)),
            pl.BlockSpec((B, 16), lambda j: (0, 0)),
            pl.BlockSpec((B, 16), lambda j: (0, 0)),
            pl.BlockSpec((1, H), lambda j: (0, 0)),
        ],
        out_specs=(pl.BlockSpec((B, 1, DEC_BL), lambda j: (0, 0, j)),
                   pl.BlockSpec((B, 1, H), lambda j: (0, 0, 0))),
        out_shape=(jax.ShapeDtypeStruct((B, 1, L), jnp.float32),
                   jax.ShapeDtypeStruct((B, 1, H), jnp.float32)),
    )(pre, W_dec, T, C, pb)

    return latents, x_hat


# speedup vs baseline: 1.3128x; 1.0173x over previous
import functools

import jax
import jax.numpy as jnp
from jax import lax
from jax.experimental import pallas as pl
from jax.experimental.pallas import tpu as pltpu
from jax.experimental.pallas import tpu_sc as plsc

B = 64
H = 768
L = 24576
K = 64

ENC_BL = 4096
DEC_BL = 4096

NV = L // 16
SPAN = 96
NSP = NV // SPAN
CAP = L + 16

_u32 = jnp.uint32
_i32 = jnp.int32


def _sortable(v):
    ub = lax.bitcast_convert_type(v, _u32)
    return jnp.where((ub >> 31) == 1, ~ub, ub | _u32(0x80000000))


def _encode_body(x_ref, pb_ref, w_ref, lb_ref, out_ref):
    xm = x_ref[...] - pb_ref[...]
    acc = lax.dot_general(
        xm, w_ref[...], (((1,), (1,)), ((), ())),
        preferred_element_type=jnp.float32)
    out_ref[...] = acc + lb_ref[...]


def _ssum(v_i32):
    return lax.reduce_sum(v_i32, axes=(0,))


def _sc_topk_fn():
    mesh = plsc.VectorSubcoreMesh(core_axis_name="c", subcore_axis_name="s")

    @functools.partial(
        pl.kernel, mesh=mesh,
        compiler_params=pltpu.CompilerParams(needs_layout_passes=False),
        out_type=(jax.ShapeDtypeStruct((B, 16), _i32),
                  jax.ShapeDtypeStruct((B, 16), _i32)),
        scratch_types=[
            pltpu.VMEM((L,), jnp.float32),
            pltpu.VMEM((L,), _u32),
            pltpu.VMEM((NSP * 16,), _u32),
            pltpu.VMEM((CAP,), _u32),
            pltpu.VMEM((CAP,), _i32),
            pltpu.VMEM((16,), _i32),
            pltpu.VMEM((16,), _i32),
            pltpu.SemaphoreType.DMA,
        ],
    )
    def sc_topk(pre_hbm, t_hbm, c_hbm, row_v, keys_v, m2_v,
                ck_v, ci_v, to_v, co_v, sem):
        c = lax.axis_index("c")
        s = lax.axis_index("s")
        wid = s * 2 + c
        r0 = wid * 2

        lanes = lax.iota(_i32, 16)

        def popcnt(mask):
            return plsc.all_reduce_population_count(mask)[0]

        pltpu.async_copy(pre_hbm.at[r0], row_v, sem).wait()

        def do_row(rr, next_r, prefetch):
            def p1_span(sp, _):
                def p1_v(i, ms):
                    m2a, m2b = ms
                    j = sp * SPAN + i * 2
                    ka = _sortable(row_v[pl.ds(j * 16, 16)])
                    kb = _sortable(row_v[pl.ds((j + 1) * 16, 16)])
                    keys_v[pl.ds(j * 16, 16)] = ka
                    keys_v[pl.ds((j + 1) * 16, 16)] = kb
                    return (jnp.maximum(m2a, ka), jnp.maximum(m2b, kb))
                z = jnp.zeros((16,), _u32)
                m2a, m2b = lax.fori_loop(0, SPAN // 2, p1_v, (z, z),
                                         unroll=4)
                m2_v[pl.ds(sp * 16, 16)] = jnp.maximum(m2a, m2b)
                return 0

            lax.fori_loop(0, NSP, p1_span, 0)

            nxt = (pltpu.async_copy(pre_hbm.at[next_r], row_v, sem)
                   if prefetch else None)

            def t0_bit(i, T):
                cand = T | (_u32(1) << (_u32(31) - i.astype(_u32)))

                def acc(j, cv):
                    return cv + plsc.all_reduce_population_count(
                        m2_v[pl.ds(j * 16, 16)] >= cand)

                cnt = lax.fori_loop(0, NSP, acc,
                                    jnp.zeros((16,), _i32), unroll=4)
                return jnp.where(cnt >= K, cand, T)

            t0v = lax.fori_loop(0, 32, t0_bit, jnp.zeros((16,), _u32))
            t0 = t0v[0]

            PB = 16

            def p2(jb, off):
                kks = []
                ms = []
                pcs = []
                for t in range(PB):
                    kk = keys_v[pl.ds((jb * PB + t) * 16, 16)]
                    m = kk >= t0
                    kks.append(kk)
                    ms.append(m)
                    pcs.append(popcnt(m))
                for t in range(PB):
                    plsc.store_compressed(ck_v.at[pl.ds(off, 16)], kks[t],
                                          mask=ms[t])
                    plsc.store_compressed(ci_v.at[pl.ds(off, 16)],
                                          lanes + (jb * PB + t) * 16,
                                          mask=ms[t])
                    off = off + pcs[t]
                return off

            ncand = lax.fori_loop(0, NV // PB, p2, _i32(0))

            ck_v[pl.ds(ncand, 16)] = jnp.zeros((16,), _u32)
            ci_v[pl.ds(ncand, 16)] = jnp.full((16,), L, _i32)
            nv = (ncand + 15) // 16

            def p3_bit(i, T):
                cand = T | (_u32(1) << (_u32(31) - i.astype(_u32)))

                def acc(j, cv):
                    return cv + plsc.all_reduce_population_count(
                        ck_v[pl.ds(j * 16, 16)] >= cand)

                cnt = lax.fori_loop(0, nv, acc, jnp.zeros((16,), _i32))
                return jnp.where(cnt >= K, cand, T)

            tkeyv = lax.fori_loop(0, 32, p3_bit, jnp.zeros((16,), _u32))
            tkey = tkeyv[0]

            def acc_gt(j, cv):
                return cv + plsc.all_reduce_population_count(
                    ck_v[pl.ds(j * 16, 16)] > tkey)

            cnt_gt = lax.fori_loop(0, nv, acc_gt,
                                   jnp.zeros((16,), _i32))[0]
            need = K - cnt_gt

            def p4(j, st):
                acc2, cidx = st
                tie = (ck_v[pl.ds(j * 16, 16)] == tkey)
                ti = tie.astype(_i32)
                cnt = _ssum(ti)
                cs = plsc.cumsum(ti)
                want = need - acc2
                m = tie & (cs == want)
                lane = lax.reduce_min(jnp.where(m, lanes, 16), axes=(0,))
                hit = (acc2 < need) & (lane < 16)
                idxv = _ssum(jnp.where(lanes == lane,
                                       ci_v[pl.ds(j * 16, 16)], 0))
                cidx = jnp.where(hit, idxv + 1, cidx)
                return (acc2 + cnt, cidx)

            _, cfin = lax.fori_loop(0, nv, p4, (_i32(0), _i32(0)))

            to_v[...] = jnp.full((16,), lax.bitcast_convert_type(tkey, _i32),
                                 _i32)
            co_v[...] = jnp.full((16,), cfin, _i32)
            pltpu.sync_copy(to_v, t_hbm.at[rr])
            pltpu.sync_copy(co_v, c_hbm.at[rr])
            return nxt

        nxt = do_row(r0, r0 + 1, True)
        nxt.wait()
        do_row(r0 + 1, r0 + 1, False)

    return sc_topk


def _decode_body(pre_ref, w_ref, t_ref, c_ref, pb_ref, lat_ref, xhat_ref):
    j = pl.program_id(0)
    pre = pre_ref[...]
    key = _sortable(pre)
    T = lax.bitcast_convert_type(t_ref[:, :1], _u32)
    C = c_ref[:, :1]
    idx = lax.broadcasted_iota(_i32, (B, DEC_BL), 1) + j * DEC_BL
    keep = (key > T) | ((key == T) & (idx < C))
    lat = jnp.where(keep, pre, 0.0)
    lat_ref[:, 0, :] = lat
    part = lax.dot_general(
        lat, w_ref[...], (((1,), (1,)), ((), ())),
        preferred_element_type=jnp.float32)

    @pl.when(j == 0)
    def _():
        xhat_ref[:, 0, :] = jnp.broadcast_to(pb_ref[...], (B, H))

    xhat_ref[:, 0, :] += part


@jax.jit
def kernel(x, W_enc, W_dec, pre_bias, latent_bias):
    x2d = x.reshape(B, H)
    pb = pre_bias.reshape(1, H)
    lb = latent_bias.reshape(1, L)

    pre = pl.pallas_call(
        _encode_body,
        grid=(L // ENC_BL,),
        in_specs=[
            pl.BlockSpec((B, H), lambda j: (0, 0)),
            pl.BlockSpec((1, H), lambda j: (0, 0)),
            pl.BlockSpec((ENC_BL, H), lambda j: (j, 0)),
            pl.BlockSpec((1, ENC_BL), lambda j: (0, j)),
        ],
        out_specs=pl.BlockSpec((B, ENC_BL), lambda j: (0, j)),
        out_shape=jax.ShapeDtypeStruct((B, L), jnp.float32),
    )(x2d, pb, W_enc, lb)

    T, C = _sc_topk_fn()(pre)

    latents, x_hat = pl.pallas_call(
        _decode_body,
        grid=(L // DEC_BL,),
        in_specs=[
            pl.BlockSpec((B, DEC_BL), lambda j: (0, j)),
            pl.BlockSpec((H, DEC_BL), lambda j: (0, j)),
            pl.BlockSpec((B, 16), lambda j: (0, 0)),
            pl.BlockSpec((B, 16), lambda j: (0, 0)),
            pl.BlockSpec((1, H), lambda j: (0, 0)),
        ],
        out_specs=(pl.BlockSpec((B, 1, DEC_BL), lambda j: (0, 0, j)),
                   pl.BlockSpec((B, 1, H), lambda j: (0, 0, 0))),
        out_shape=(jax.ShapeDtypeStruct((B, 1, L), jnp.float32),
                   jax.ShapeDtypeStruct((B, 1, H), jnp.float32)),
    )(pre, W_dec, T, C, pb)

    return latents, x_hat
